# Initial kernel scaffold; baseline (speedup 1.0000x reference)
import jax
import jax.numpy as jnp
from jax.experimental import pallas as pl


def kernel(x, edge_index, W1, b1, W2, b2, gamma, beta):
    raise NotImplementedError("write your pallas kernel here")


# speedup vs baseline: 11.9883x; 11.9883x over previous
import functools

import jax
import jax.numpy as jnp
from jax import lax
from jax.experimental import pallas as pl
from jax.experimental.pallas import tpu as pltpu
from jax.experimental.pallas import tpu_sc as plsc

_N = 10000
_E = 320000
_D = 128
_DH = 64
_EPS = 1e-5

_NC = 2
_NS = 16
_RP = 10240
_CW = 128
_CHUNKS = 160
_EPT = _CHUNKS * _CW
_EPAD = _NS * _EPT
_STRIPE = _RP // _NS
_DW = 16

_sc_mesh = plsc.VectorSubcoreMesh(
    core_axis_name="c", subcore_axis_name="s", num_cores=_NC, num_subcores=_NS
)


def _fill_vmem(ref, rows, width, value):
    def body(i, _):
        for k in range(width // 16):
            ref[i, pl.ds(k * 16, 16)] = jnp.full((16,), value, jnp.float32)
        return 0
    lax.fori_loop(0, rows, body, 0)


@functools.partial(
    pl.kernel,
    out_type=jax.ShapeDtypeStruct((_NC, _RP, _DW), jnp.float32),
    mesh=_sc_mesh,
    scratch_types=[
        pltpu.VMEM((_CHUNKS // 2, _CW), jnp.int32),
        pltpu.VMEM((_CW, _DW), jnp.float32),
        pltpu.VMEM_SHARED((_RP, _DW), jnp.float32),
    ],
)
def _sc_degree(dst_hbm, out_hbm, dst_v, buf_v, deg_sh):
    c = lax.axis_index("c")
    s = lax.axis_index("s")
    _fill_vmem(buf_v, _CW, _DW, 0.0)
    for i in range(_STRIPE // _CW):
        pltpu.sync_copy(buf_v, deg_sh.at[pl.ds(s * _STRIPE + i * _CW, _CW)])
    plsc.subcore_barrier()
    pltpu.sync_copy(dst_hbm.at[s, pl.ds(c * (_CHUNKS // 2), _CHUNKS // 2)],
                    dst_v)
    _fill_vmem(buf_v, _CW, _DW, 1.0)

    def body(j, _):
        pltpu.sync_copy(buf_v, deg_sh.at[dst_v.at[j]], add=True)
        return 0
    lax.fori_loop(0, _CHUNKS // 2, body, 0)
    plsc.subcore_barrier()
    for i in range(_STRIPE // _CW):
        pltpu.sync_copy(deg_sh.at[pl.ds(s * _STRIPE + i * _CW, _CW)], buf_v)
        pltpu.sync_copy(buf_v, out_hbm.at[c, pl.ds(s * _STRIPE + i * _CW, _CW)])


@functools.partial(
    pl.kernel,
    out_type=jax.ShapeDtypeStruct((_NC, _RP, _DH), jnp.float32),
    mesh=_sc_mesh,
    compiler_params=pltpu.CompilerParams(use_tc_tiling_on_sc=False),
    scratch_types=[
        pltpu.VMEM((_CHUNKS, _CW), jnp.int32),
        pltpu.VMEM((_CHUNKS, _CW), jnp.int32),
        pltpu.VMEM((_CW, _DH), jnp.float32),
        pltpu.VMEM((_CW, _DH), jnp.float32),
        pltpu.VMEM_SHARED((_RP, _DH), jnp.float32),
        pltpu.SemaphoreType.DMA,
    ],
)
def _sc_scatter(pre_hbm, src_hbm, dst_hbm, out_hbm,
                src_v, dst_v, rows_v, buf_v, acc_sh, sem):
    c = lax.axis_index("c")
    s = lax.axis_index("s")
    _fill_vmem(buf_v, _CW, _DH, 0.0)
    for i in range(_STRIPE // _CW):
        pltpu.sync_copy(buf_v, acc_sh.at[pl.ds(s * _STRIPE + i * _CW, _CW)])
    plsc.subcore_barrier()
    pltpu.sync_copy(src_hbm.at[s], src_v)
    pltpu.sync_copy(dst_hbm.at[s], dst_v)

    def body(j, _):
        pltpu.async_copy(pre_hbm.at[c].at[src_v.at[j]], rows_v, sem).wait()
        pltpu.sync_copy(rows_v, acc_sh.at[dst_v.at[j]], add=True)
        return 0
    lax.fori_loop(0, _CHUNKS, body, 0)
    plsc.subcore_barrier()
    for i in range(_STRIPE // _CW):
        pltpu.sync_copy(acc_sh.at[pl.ds(s * _STRIPE + i * _CW, _CW)], buf_v)
        pltpu.sync_copy(buf_v, out_hbm.at[c, pl.ds(s * _STRIPE + i * _CW, _CW)])


def _tc1_body(x_ref, w1_ref, deg_ref, pre_ref, dinv_ref):
    deg = deg_ref[0] + deg_ref[1] + 1.0
    dinv = lax.rsqrt(deg)
    h = jnp.dot(x_ref[...], w1_ref[...], preferred_element_type=jnp.float32)
    pre = h * dinv[:, 0:1]
    pre_ref[0] = pre[:, :_DH]
    pre_ref[1] = pre[:, _DH:]
    dinv_ref[...] = dinv


_tc1 = pl.pallas_call(
    _tc1_body,
    out_shape=[
        jax.ShapeDtypeStruct((_NC, _RP, _DH), jnp.float32),
        jax.ShapeDtypeStruct((_RP, _DW), jnp.float32),
    ],
)


def _tc2_body(pre_ref, agg_ref, dinv_ref, b1_ref, w2_ref, pre2_ref):
    dinv = dinv_ref[...][:, 0:1]
    tot = jnp.concatenate([agg_ref[0] + pre_ref[0], agg_ref[1] + pre_ref[1]],
                          axis=-1)
    z = tot * dinv + b1_ref[...]
    hr = jnp.maximum(z, 0.0)
    h2 = jnp.dot(hr, w2_ref[...], preferred_element_type=jnp.float32)
    pre2 = h2 * dinv
    pre2_ref[0] = pre2[:, :_DH]
    pre2_ref[1] = pre2[:, _DH:]


_tc2 = pl.pallas_call(
    _tc2_body,
    out_shape=jax.ShapeDtypeStruct((_NC, _RP, _DH), jnp.float32),
)


def _tc3_body(pre_ref, agg_ref, dinv_ref, b2_ref, g_ref, bt_ref, out_ref):
    dinv = dinv_ref[...][:, 0:1]
    tot = jnp.concatenate([agg_ref[0] + pre_ref[0], agg_ref[1] + pre_ref[1]],
                          axis=-1)
    z = tot * dinv + b2_ref[...]
    r = jnp.maximum(z, 0.0)
    rowid = lax.broadcasted_iota(jnp.int32, (_RP, 1), 0)
    mask = rowid < _N
    rm = jnp.where(mask, r, 0.0)
    mean = jnp.sum(rm, axis=0, keepdims=True) * (1.0 / _N)
    dev = jnp.where(mask, r - mean, 0.0)
    var = jnp.sum(dev * dev, axis=0, keepdims=True) * (1.0 / _N)
    out_ref[...] = (r - mean) * lax.rsqrt(var + _EPS) * g_ref[...] + bt_ref[...]


_tc3 = pl.pallas_call(
    _tc3_body,
    out_shape=jax.ShapeDtypeStruct((_RP, _D), jnp.float32),
)


def kernel(x, edge_index, W1, b1, W2, b2, gamma, beta):
    src = edge_index[0]
    dst = edge_index[1]
    npad = _EPAD - _E
    src_p = jnp.concatenate(
        [src, jnp.zeros((npad,), jnp.int32)]).reshape(_NS, _CHUNKS, _CW)
    dst_p = jnp.concatenate(
        [dst, jnp.full((npad,), _N, jnp.int32)]).reshape(_NS, _CHUNKS, _CW)
    x_pad = jnp.zeros((_RP, _D), jnp.float32).at[:_N].set(x)

    deg = _sc_degree(dst_p)
    pre1, dinv = _tc1(x_pad, W1, deg)
    agg1 = _sc_scatter(pre1, src_p, dst_p)
    pre2 = _tc2(pre1, agg1, dinv, b1.reshape(1, _D), W2)
    agg2 = _sc_scatter(pre2, src_p, dst_p)
    out = _tc3(pre2, agg2, dinv, b2.reshape(1, _D),
               gamma.reshape(1, _D), beta.reshape(1, _D))
    return out[:_N]


# speedup vs baseline: 15.6004x; 1.3013x over previous
import functools

import jax
import jax.numpy as jnp
from jax import lax
from jax.experimental import pallas as pl
from jax.experimental.pallas import tpu as pltpu
from jax.experimental.pallas import tpu_sc as plsc

_N = 10000
_E = 320000
_D = 128
_DH = 64
_EPS = 1e-5

_NC = 2
_NS = 16
_RP = 10240
_CW = 128
_CHUNKS = 160
_EPT = _CHUNKS * _CW
_EPAD = _NS * _EPT
_STRIPE = _RP // _NS
_DW = 16

_sc_mesh = plsc.VectorSubcoreMesh(
    core_axis_name="c", subcore_axis_name="s", num_cores=_NC, num_subcores=_NS
)


def _fill_vmem(ref, rows, width, value):
    def body(i, _):
        for k in range(width // 16):
            ref[i, pl.ds(k * 16, 16)] = jnp.full((16,), value, jnp.float32)
        return 0
    lax.fori_loop(0, rows, body, 0)


@functools.partial(
    pl.kernel,
    out_type=jax.ShapeDtypeStruct((_NC, _RP, _DW), jnp.float32),
    mesh=_sc_mesh,
    scratch_types=[
        pltpu.VMEM((_CHUNKS // 2, _CW), jnp.int32),
        pltpu.VMEM((_CW, _DW), jnp.float32),
        pltpu.VMEM_SHARED((_RP, _DW), jnp.float32),
    ],
)
def _sc_degree(dst_hbm, out_hbm, dst_v, buf_v, deg_sh):
    c = lax.axis_index("c")
    s = lax.axis_index("s")
    _fill_vmem(buf_v, _CW, _DW, 0.0)
    for i in range(_STRIPE // _CW):
        pltpu.sync_copy(buf_v, deg_sh.at[pl.ds(s * _STRIPE + i * _CW, _CW)])
    plsc.subcore_barrier()
    pltpu.sync_copy(dst_hbm.at[s, pl.ds(c * (_CHUNKS // 2), _CHUNKS // 2)],
                    dst_v)
    _fill_vmem(buf_v, _CW, _DW, 1.0)

    def body(j, _):
        pltpu.sync_copy(buf_v, deg_sh.at[dst_v.at[j]], add=True)
        return 0
    lax.fori_loop(0, _CHUNKS // 2, body, 0)
    plsc.subcore_barrier()
    for i in range(_STRIPE // _CW):
        pltpu.sync_copy(deg_sh.at[pl.ds(s * _STRIPE + i * _CW, _CW)], buf_v)
        pltpu.sync_copy(buf_v, out_hbm.at[c, pl.ds(s * _STRIPE + i * _CW, _CW)])


@functools.partial(
    pl.kernel,
    out_type=jax.ShapeDtypeStruct((_NC, _RP, _DH), jnp.float32),
    mesh=_sc_mesh,
    compiler_params=pltpu.CompilerParams(use_tc_tiling_on_sc=False),
    scratch_types=[
        pltpu.VMEM((_CHUNKS, _CW), jnp.int32),
        pltpu.VMEM((_CHUNKS, _CW), jnp.int32),
        pltpu.VMEM((_CW, _DH), jnp.float32),
        pltpu.VMEM((_CW, _DH), jnp.float32),
        pltpu.VMEM((_CW, _DH), jnp.float32),
        pltpu.VMEM_SHARED((_RP, _DH), jnp.float32),
        pltpu.SemaphoreType.DMA,
        pltpu.SemaphoreType.DMA,
    ],
)
def _sc_scatter(pre_hbm, src_hbm, dst_hbm, out_hbm,
                src_v, dst_v, rows0_v, rows1_v, buf_v, acc_sh, sem0, sem1):
    c = lax.axis_index("c")
    s = lax.axis_index("s")
    _fill_vmem(buf_v, _CW, _DH, 0.0)
    for i in range(_STRIPE // _CW):
        pltpu.sync_copy(buf_v, acc_sh.at[pl.ds(s * _STRIPE + i * _CW, _CW)])
    plsc.subcore_barrier()
    pltpu.sync_copy(src_hbm.at[s], src_v)
    pltpu.sync_copy(dst_hbm.at[s], dst_v)

    rows = (rows0_v, rows1_v)
    sems = (sem0, sem1)
    pltpu.async_copy(pre_hbm.at[c].at[src_v.at[0]], rows0_v, sem0)

    def body(jo, _):
        for b in range(2):
            j = jo * 2 + b
            nb = 1 - b

            @pl.when(j < _CHUNKS - 1)
            def _():
                pltpu.async_copy(pre_hbm.at[c].at[src_v.at[j + 1]],
                                 rows[nb], sems[nb])
            pltpu.make_async_copy(pre_hbm.at[c].at[src_v.at[j]],
                                  rows[b], sems[b]).wait()
            pltpu.sync_copy(rows[b], acc_sh.at[dst_v.at[j]], add=True)
        return 0
    lax.fori_loop(0, _CHUNKS // 2, body, 0)
    plsc.subcore_barrier()
    for i in range(_STRIPE // _CW):
        pltpu.sync_copy(acc_sh.at[pl.ds(s * _STRIPE + i * _CW, _CW)], buf_v)
        pltpu.sync_copy(buf_v, out_hbm.at[c, pl.ds(s * _STRIPE + i * _CW, _CW)])


def _tc1_body(x_ref, w1_ref, deg_ref, pre_ref, dinv_ref):
    deg = deg_ref[0] + deg_ref[1] + 1.0
    dinv = lax.rsqrt(deg)
    h = jnp.dot(x_ref[...], w1_ref[...], preferred_element_type=jnp.float32)
    pre = h * dinv[:, 0:1]
    pre_ref[0] = pre[:, :_DH]
    pre_ref[1] = pre[:, _DH:]
    dinv_ref[...] = dinv


_tc1 = pl.pallas_call(
    _tc1_body,
    out_shape=[
        jax.ShapeDtypeStruct((_NC, _RP, _DH), jnp.float32),
        jax.ShapeDtypeStruct((_RP, _DW), jnp.float32),
    ],
)


def _tc2_body(pre_ref, agg_ref, dinv_ref, b1_ref, w2_ref, pre2_ref):
    dinv = dinv_ref[...][:, 0:1]
    tot = jnp.concatenate([agg_ref[0] + pre_ref[0], agg_ref[1] + pre_ref[1]],
                          axis=-1)
    z = tot * dinv + b1_ref[...]
    hr = jnp.maximum(z, 0.0)
    h2 = jnp.dot(hr, w2_ref[...], preferred_element_type=jnp.float32)
    pre2 = h2 * dinv
    pre2_ref[0] = pre2[:, :_DH]
    pre2_ref[1] = pre2[:, _DH:]


_tc2 = pl.pallas_call(
    _tc2_body,
    out_shape=jax.ShapeDtypeStruct((_NC, _RP, _DH), jnp.float32),
)


def _tc3_body(pre_ref, agg_ref, dinv_ref, b2_ref, g_ref, bt_ref, out_ref):
    dinv = dinv_ref[...][:, 0:1]
    tot = jnp.concatenate([agg_ref[0] + pre_ref[0], agg_ref[1] + pre_ref[1]],
                          axis=-1)
    z = tot * dinv + b2_ref[...]
    r = jnp.maximum(z, 0.0)
    rowid = lax.broadcasted_iota(jnp.int32, (_RP, 1), 0)
    mask = rowid < _N
    rm = jnp.where(mask, r, 0.0)
    mean = jnp.sum(rm, axis=0, keepdims=True) * (1.0 / _N)
    dev = jnp.where(mask, r - mean, 0.0)
    var = jnp.sum(dev * dev, axis=0, keepdims=True) * (1.0 / _N)
    out_ref[...] = (r - mean) * lax.rsqrt(var + _EPS) * g_ref[...] + bt_ref[...]


_tc3 = pl.pallas_call(
    _tc3_body,
    out_shape=jax.ShapeDtypeStruct((_RP, _D), jnp.float32),
)


def kernel(x, edge_index, W1, b1, W2, b2, gamma, beta):
    src = edge_index[0]
    dst = edge_index[1]
    npad = _EPAD - _E
    src_p = jnp.concatenate(
        [src, jnp.zeros((npad,), jnp.int32)]).reshape(_NS, _CHUNKS, _CW)
    dst_p = jnp.concatenate(
        [dst, jnp.full((npad,), _N, jnp.int32)]).reshape(_NS, _CHUNKS, _CW)
    x_pad = jnp.zeros((_RP, _D), jnp.float32).at[:_N].set(x)

    deg = _sc_degree(dst_p)
    pre1, dinv = _tc1(x_pad, W1, deg)
    agg1 = _sc_scatter(pre1, src_p, dst_p)
    pre2 = _tc2(pre1, agg1, dinv, b1.reshape(1, _D), W2)
    agg2 = _sc_scatter(pre2, src_p, dst_p)
    out = _tc3(pre2, agg2, dinv, b2.reshape(1, _D),
               gamma.reshape(1, _D), beta.reshape(1, _D))
    return out[:_N]


# speedup vs baseline: 15.6809x; 1.0052x over previous
import functools

import jax
import jax.numpy as jnp
from jax import lax
from jax.experimental import pallas as pl
from jax.experimental.pallas import tpu as pltpu
from jax.experimental.pallas import tpu_sc as plsc

_N = 10000
_E = 320000
_D = 128
_DH = 64
_EPS = 1e-5

_NC = 2
_NS = 16
_RP = 10240
_CW = 128
_CHUNKS = 160
_EPT = _CHUNKS * _CW
_EPAD = _NS * _EPT
_STRIPE = _RP // _NS
_DW = 16

_sc_mesh = plsc.VectorSubcoreMesh(
    core_axis_name="c", subcore_axis_name="s", num_cores=_NC, num_subcores=_NS
)


def _fill_vmem(ref, rows, width, value):
    def body(i, _):
        for k in range(width // 16):
            ref[i, pl.ds(k * 16, 16)] = jnp.full((16,), value, jnp.float32)
        return 0
    lax.fori_loop(0, rows, body, 0)


@functools.partial(
    pl.kernel,
    out_type=jax.ShapeDtypeStruct((_NC, _RP, _DW), jnp.float32),
    mesh=_sc_mesh,
    scratch_types=[
        pltpu.VMEM((_CHUNKS // 2, _CW), jnp.int32),
        pltpu.VMEM((_CW, _DW), jnp.float32),
        pltpu.VMEM_SHARED((_RP, _DW), jnp.float32),
    ],
)
def _sc_degree(dst_hbm, out_hbm, dst_v, buf_v, deg_sh):
    c = lax.axis_index("c")
    s = lax.axis_index("s")
    _fill_vmem(buf_v, _CW, _DW, 0.0)
    for i in range(_STRIPE // _CW):
        pltpu.sync_copy(buf_v, deg_sh.at[pl.ds(s * _STRIPE + i * _CW, _CW)])
    plsc.subcore_barrier()
    pltpu.sync_copy(dst_hbm.at[s, pl.ds(c * (_CHUNKS // 2), _CHUNKS // 2)],
                    dst_v)
    _fill_vmem(buf_v, _CW, _DW, 1.0)

    def body(j, _):
        pltpu.sync_copy(buf_v, deg_sh.at[dst_v.at[j]], add=True)
        return 0
    lax.fori_loop(0, _CHUNKS // 2, body, 0)
    plsc.subcore_barrier()
    for i in range(_STRIPE // _CW):
        pltpu.sync_copy(deg_sh.at[pl.ds(s * _STRIPE + i * _CW, _CW)], buf_v)
        pltpu.sync_copy(buf_v, out_hbm.at[c, pl.ds(s * _STRIPE + i * _CW, _CW)])


@functools.partial(
    pl.kernel,
    out_type=jax.ShapeDtypeStruct((_NC, _RP, _DH), jnp.float32),
    mesh=_sc_mesh,
    compiler_params=pltpu.CompilerParams(use_tc_tiling_on_sc=False),
    scratch_types=[
        pltpu.VMEM((_CHUNKS, _CW), jnp.int32),
        pltpu.VMEM((_CHUNKS, _CW), jnp.int32),
        [pltpu.VMEM((_CW, _DH), jnp.float32)] * 4,
        pltpu.VMEM((_CW, _DH), jnp.float32),
        pltpu.VMEM_SHARED((_RP, _DH), jnp.float32),
        [pltpu.SemaphoreType.DMA] * 4,
        [pltpu.SemaphoreType.DMA] * 4,
    ],
)
def _sc_scatter(pre_hbm, src_hbm, dst_hbm, out_hbm,
                src_v, dst_v, rows, buf_v, acc_sh, gsem, ssem):
    c = lax.axis_index("c")
    s = lax.axis_index("s")
    _fill_vmem(buf_v, _CW, _DH, 0.0)
    for i in range(_STRIPE // _CW):
        pltpu.sync_copy(buf_v, acc_sh.at[pl.ds(s * _STRIPE + i * _CW, _CW)])
    plsc.subcore_barrier()
    pltpu.sync_copy(src_hbm.at[s], src_v)
    pltpu.sync_copy(dst_hbm.at[s], dst_v)

    _NB = 4
    for b in range(_NB):
        pltpu.async_copy(pre_hbm.at[c].at[src_v.at[b]], rows[b], gsem[b])

    def body(jo, _):
        for b in range(_NB):
            j = jo * _NB + b
            pltpu.make_async_copy(pre_hbm.at[c].at[src_v.at[j]],
                                  rows[b], gsem[b]).wait()
            pltpu.async_copy(rows[b], acc_sh.at[dst_v.at[j]], ssem[b],
                             add=True)
        for b in range(_NB):
            j = jo * _NB + b
            pltpu.make_async_copy(rows[b], acc_sh.at[dst_v.at[j]],
                                  ssem[b]).wait()

            @pl.when(j + _NB < _CHUNKS)
            def _():
                pltpu.async_copy(pre_hbm.at[c].at[src_v.at[j + _NB]],
                                 rows[b], gsem[b])
        return 0
    lax.fori_loop(0, _CHUNKS // _NB, body, 0)
    plsc.subcore_barrier()
    for i in range(_STRIPE // _CW):
        pltpu.sync_copy(acc_sh.at[pl.ds(s * _STRIPE + i * _CW, _CW)], buf_v)
        pltpu.sync_copy(buf_v, out_hbm.at[c, pl.ds(s * _STRIPE + i * _CW, _CW)])


def _tc1_body(x_ref, w1_ref, deg_ref, pre_ref, dinv_ref):
    deg = deg_ref[0] + deg_ref[1] + 1.0
    dinv = lax.rsqrt(deg)
    h = jnp.dot(x_ref[...], w1_ref[...], preferred_element_type=jnp.float32)
    pre = h * dinv[:, 0:1]
    pre_ref[0] = pre[:, :_DH]
    pre_ref[1] = pre[:, _DH:]
    dinv_ref[...] = dinv


_tc1 = pl.pallas_call(
    _tc1_body,
    out_shape=[
        jax.ShapeDtypeStruct((_NC, _RP, _DH), jnp.float32),
        jax.ShapeDtypeStruct((_RP, _DW), jnp.float32),
    ],
)


def _tc2_body(pre_ref, agg_ref, dinv_ref, b1_ref, w2_ref, pre2_ref):
    dinv = dinv_ref[...][:, 0:1]
    tot = jnp.concatenate([agg_ref[0] + pre_ref[0], agg_ref[1] + pre_ref[1]],
                          axis=-1)
    z = tot * dinv + b1_ref[...]
    hr = jnp.maximum(z, 0.0)
    h2 = jnp.dot(hr, w2_ref[...], preferred_element_type=jnp.float32)
    pre2 = h2 * dinv
    pre2_ref[0] = pre2[:, :_DH]
    pre2_ref[1] = pre2[:, _DH:]


_tc2 = pl.pallas_call(
    _tc2_body,
    out_shape=jax.ShapeDtypeStruct((_NC, _RP, _DH), jnp.float32),
)


def _tc3_body(pre_ref, agg_ref, dinv_ref, b2_ref, g_ref, bt_ref, out_ref):
    dinv = dinv_ref[...][:, 0:1]
    tot = jnp.concatenate([agg_ref[0] + pre_ref[0], agg_ref[1] + pre_ref[1]],
                          axis=-1)
    z = tot * dinv + b2_ref[...]
    r = jnp.maximum(z, 0.0)
    rowid = lax.broadcasted_iota(jnp.int32, (_RP, 1), 0)
    mask = rowid < _N
    rm = jnp.where(mask, r, 0.0)
    mean = jnp.sum(rm, axis=0, keepdims=True) * (1.0 / _N)
    dev = jnp.where(mask, r - mean, 0.0)
    var = jnp.sum(dev * dev, axis=0, keepdims=True) * (1.0 / _N)
    out_ref[...] = (r - mean) * lax.rsqrt(var + _EPS) * g_ref[...] + bt_ref[...]


_tc3 = pl.pallas_call(
    _tc3_body,
    out_shape=jax.ShapeDtypeStruct((_RP, _D), jnp.float32),
)


def kernel(x, edge_index, W1, b1, W2, b2, gamma, beta):
    src = edge_index[0]
    dst = edge_index[1]
    npad = _EPAD - _E
    src_p = jnp.concatenate(
        [src, jnp.zeros((npad,), jnp.int32)]).reshape(_NS, _CHUNKS, _CW)
    dst_p = jnp.concatenate(
        [dst, jnp.full((npad,), _N, jnp.int32)]).reshape(_NS, _CHUNKS, _CW)
    x_pad = jnp.zeros((_RP, _D), jnp.float32).at[:_N].set(x)

    deg = _sc_degree(dst_p)
    pre1, dinv = _tc1(x_pad, W1, deg)
    agg1 = _sc_scatter(pre1, src_p, dst_p)
    pre2 = _tc2(pre1, agg1, dinv, b1.reshape(1, _D), W2)
    agg2 = _sc_scatter(pre2, src_p, dst_p)
    out = _tc3(pre2, agg2, dinv, b2.reshape(1, _D),
               gamma.reshape(1, _D), beta.reshape(1, _D))
    return out[:_N]
